# Initial kernel scaffold; baseline (speedup 1.0000x reference)
import jax
import jax.numpy as jnp
from jax.experimental import pallas as pl


def kernel(x, embedding, ema_weight, ema_count):
    raise NotImplementedError("write your pallas kernel here")


# speedup vs baseline: 138.0248x; 138.0248x over previous
import functools

import jax
import jax.numpy as jnp
from jax import lax
from jax.experimental import pallas as pl
from jax.experimental.pallas import tpu as pltpu

_N = 4
_M = 1024
_D = 64
_L = 16
_B = 1024
_T = _B * _L
_DECAY = 0.999
_EPS = 1e-05
_COMMIT = 0.05

_TB = 512
_NTB = _T // _TB


def _dist_argmin_body(x_ref, emb_ref, en_ref, xn_ref, idx_ref, fidx_ref, cnt_ref):
    n = pl.program_id(0)
    t = pl.program_id(1)
    x = x_ref[0]
    e = emb_ref[0]
    scores = lax.dot_general(
        x, e, (((1,), (1,)), ((), ())),
        preferred_element_type=jnp.float32)
    to_add = en_ref[0] + xn_ref[0]
    dist = to_add - 2.0 * scores
    mn = jnp.min(dist, axis=1, keepdims=True)
    iota = lax.broadcasted_iota(jnp.int32, (_TB, _M), 1)
    idx = jnp.min(jnp.where(dist == mn, iota, _M), axis=1, keepdims=True)
    idx_ref[0] = idx
    fidx_ref[0] = idx + n * _M
    part = jnp.sum((idx == iota).astype(jnp.float32), axis=0, keepdims=True)

    @pl.when(t == 0)
    def _():
        cnt_ref[0] = part

    @pl.when(t != 0)
    def _():
        cnt_ref[0] = cnt_ref[0] + part


def _dist_argmin(x_flat, embedding, e_norm, x_norm):
    grid = (_N, _NTB)
    out_shapes = [
        jax.ShapeDtypeStruct((_N * _NTB, _TB, 1), jnp.int32),
        jax.ShapeDtypeStruct((_N * _NTB, _TB, 1), jnp.int32),
        jax.ShapeDtypeStruct((_N, 1, _M), jnp.float32),
    ]
    return pl.pallas_call(
        _dist_argmin_body,
        grid=grid,
        in_specs=[
            pl.BlockSpec((1, _TB, _D), lambda n, t: (n, t, 0)),
            pl.BlockSpec((1, _M, _D), lambda n, t: (n, 0, 0)),
            pl.BlockSpec((1, 1, _M), lambda n, t: (n, 0, 0)),
            pl.BlockSpec((1, _TB, 1), lambda n, t: (n, t, 0)),
        ],
        out_specs=[
            pl.BlockSpec((1, _TB, 1), lambda n, t: (n * _NTB + t, 0, 0)),
            pl.BlockSpec((1, _TB, 1), lambda n, t: (n * _NTB + t, 0, 0)),
            pl.BlockSpec((1, 1, _M), lambda n, t: (n, 0, 0)),
        ],
        out_shape=out_shapes,
        compiler_params=pltpu.CompilerParams(
            dimension_semantics=("arbitrary", "arbitrary")),
    )(x_flat, embedding, e_norm, x_norm)


def kernel(x, embedding, ema_weight, ema_count):
    bs = x.shape[0]
    xr = x.reshape(bs, _N, _D, _L)
    x_flat = jnp.transpose(xr, (1, 0, 3, 2)).reshape(_N, bs * _L, _D)
    e_norm = jnp.sum(embedding ** 2, axis=2)[:, None, :]
    x_norm = jnp.sum(x_flat ** 2, axis=2, keepdims=True)

    idx3, fidx3, counts3 = _dist_argmin(x_flat, embedding, e_norm, x_norm)
    indices = idx3.reshape(_N, _T)
    fidx = fidx3.reshape(_N * _T)
    counts = counts3.reshape(_N, _M)

    emb_flat = embedding.reshape(_N * _M, _D)
    quant = jnp.take(emb_flat, fidx, axis=0)
    x_rows = x_flat.reshape(_N * _T, _D)
    dw = jax.ops.segment_sum(x_rows, fidx, num_segments=_N * _M)

    new_ema_count = _DECAY * ema_count + (1.0 - _DECAY) * counts
    nsum = jnp.sum(new_ema_count, axis=-1, keepdims=True)
    new_ema_count = (new_ema_count + _EPS) / (nsum + _M * _EPS) * nsum
    new_ema_weight = _DECAY * ema_weight + (1.0 - _DECAY) * dw.reshape(_N, _M, _D)
    new_emb = new_ema_weight / new_ema_count[..., None]

    eq_rows = jnp.take(new_emb.reshape(_N * _M, _D), fidx, axis=0)

    loss = _COMMIT * jnp.mean((x_rows - quant) ** 2)
    avg_probs = counts / float(_T)
    perplexity = jnp.exp(
        -jnp.sum(avg_probs * jnp.log(avg_probs + 1e-10), axis=-1)).sum()

    z_q = jnp.transpose(quant.reshape(_N, bs, _L, _D), (1, 0, 3, 2)).reshape(bs, _N * _D * _L)
    encodings_q = jnp.transpose(eq_rows.reshape(_N, bs, _L, _D), (1, 0, 3, 2)).reshape(bs, _N * _D, _L, 1)
    indices_out = jnp.transpose(indices.reshape(_N, bs, _L), (1, 0, 2))[..., None]
    return (z_q, loss, perplexity, indices_out, encodings_q)


# speedup vs baseline: 230.7429x; 1.6718x over previous
import functools

import jax
import jax.numpy as jnp
from jax import lax
from jax.experimental import pallas as pl
from jax.experimental.pallas import tpu as pltpu
from jax.experimental.pallas import tpu_sc as plsc

_N = 4
_M = 1024
_D = 64
_L = 16
_B = 1024
_T = _B * _L
_NT = _N * _T
_NM = _N * _M
_DECAY = 0.999
_EPS = 1e-05
_COMMIT = 0.05

_TB = 512
_NTB = _T // _TB

_NC = 2
_NS = 16
_CH = 128
_ROWS_PER_TILE = _NT // (_NC * _NS)
_NCHUNK = _ROWS_PER_TILE // _CH


def _dist_argmin_body(x_ref, emb_ref, en_ref, xn_ref, idx_ref, fidx_ref, cnt_ref):
    n = pl.program_id(0)
    t = pl.program_id(1)
    x = x_ref[0]
    e = emb_ref[0]
    scores = lax.dot_general(
        x, e, (((1,), (1,)), ((), ())),
        preferred_element_type=jnp.float32)
    to_add = en_ref[0] + xn_ref[0]
    dist = to_add - 2.0 * scores
    mn = jnp.min(dist, axis=1, keepdims=True)
    iota = lax.broadcasted_iota(jnp.int32, (_TB, _M), 1)
    idx = jnp.min(jnp.where(dist == mn, iota, _M), axis=1, keepdims=True)
    idx_ref[0] = idx
    fidx_ref[0] = idx + n * _M
    part = jnp.sum((idx == iota).astype(jnp.float32), axis=0, keepdims=True)

    @pl.when(t == 0)
    def _():
        cnt_ref[0] = part

    @pl.when(t != 0)
    def _():
        cnt_ref[0] = cnt_ref[0] + part


def _dist_argmin(x_flat, embedding, e_norm, x_norm):
    out_shapes = [
        jax.ShapeDtypeStruct((_N * _NTB, _TB, 1), jnp.int32),
        jax.ShapeDtypeStruct((_N * _NTB, _TB, 1), jnp.int32),
        jax.ShapeDtypeStruct((_N, 1, _M), jnp.float32),
    ]
    return pl.pallas_call(
        _dist_argmin_body,
        grid=(_N, _NTB),
        in_specs=[
            pl.BlockSpec((1, _TB, _D), lambda n, t: (n, t, 0)),
            pl.BlockSpec((1, _M, _D), lambda n, t: (n, 0, 0)),
            pl.BlockSpec((1, 1, _M), lambda n, t: (n, 0, 0)),
            pl.BlockSpec((1, _TB, 1), lambda n, t: (n, t, 0)),
        ],
        out_specs=[
            pl.BlockSpec((1, _TB, 1), lambda n, t: (n * _NTB + t, 0, 0)),
            pl.BlockSpec((1, _TB, 1), lambda n, t: (n * _NTB + t, 0, 0)),
            pl.BlockSpec((1, 1, _M), lambda n, t: (n, 0, 0)),
        ],
        out_shape=out_shapes,
        compiler_params=pltpu.CompilerParams(
            dimension_semantics=("arbitrary", "arbitrary")),
    )(x_flat, embedding, e_norm, x_norm)


def _sc_gather_scatter_body(idx_hbm, x_hbm, emb_hbm, zero_hbm,
                            q_out, dw_out,
                            idx_v, qrows, xrows, dwsh, sem):
    c = lax.axis_index("c")
    s = lax.axis_index("s")
    gbase = pl.multiple_of(c * (_NM // _NC) + s * (_NM // (_NC * _NS)), 8)
    pltpu.sync_copy(zero_hbm, dwsh.at[pl.ds(gbase, _NM // (_NC * _NS))])
    plsc.subcore_barrier()

    base = c * (_NT // _NC) + s * _ROWS_PER_TILE
    pltpu.sync_copy(idx_hbm.at[pl.ds(pl.multiple_of(base // _CH, 8), _NCHUNK)],
                    idx_v)
    for j in range(_NCHUNK):
        tok = pl.multiple_of(base + j * _CH, 8)
        pltpu.async_copy(emb_hbm.at[idx_v.at[j]], qrows, sem).wait()
        pltpu.sync_copy(qrows, q_out.at[pl.ds(tok, _CH)])
        pltpu.sync_copy(x_hbm.at[pl.ds(tok, _CH)], xrows)
        pltpu.sync_copy(xrows, dwsh.at[idx_v.at[j]], add=True)
    plsc.subcore_barrier()
    pltpu.sync_copy(dwsh.at[pl.ds(gbase, _NM // (_NC * _NS))],
                    dw_out.at[pl.ds(gbase, _NM // (_NC * _NS))])


_sc_gather_scatter = functools.partial(
    pl.kernel,
    _sc_gather_scatter_body,
    out_type=[
        jax.ShapeDtypeStruct((_NT, _D), jnp.float32),
        jax.ShapeDtypeStruct((_NM, _D), jnp.float32),
    ],
    mesh=plsc.VectorSubcoreMesh(core_axis_name="c", subcore_axis_name="s"),
    compiler_params=pltpu.CompilerParams(use_tc_tiling_on_sc=False),
    scratch_types=[
        pltpu.VMEM((_NCHUNK, _CH), jnp.int32),
        pltpu.VMEM((_CH, _D), jnp.float32),
        pltpu.VMEM((_CH, _D), jnp.float32),
        pltpu.VMEM_SHARED((_NM, _D), jnp.float32),
        pltpu.SemaphoreType.DMA,
    ],
)()


def _sc_gather2_body(idx_hbm, emb_hbm, q_out, idx_v, qrows, sem):
    c = lax.axis_index("c")
    s = lax.axis_index("s")
    base = c * (_NT // _NC) + s * _ROWS_PER_TILE
    pltpu.sync_copy(idx_hbm.at[pl.ds(pl.multiple_of(base // _CH, 8), _NCHUNK)],
                    idx_v)
    for j in range(_NCHUNK):
        tok = pl.multiple_of(base + j * _CH, 8)
        pltpu.async_copy(emb_hbm.at[idx_v.at[j]], qrows, sem).wait()
        pltpu.sync_copy(qrows, q_out.at[pl.ds(tok, _CH)])


_sc_gather2 = functools.partial(
    pl.kernel,
    _sc_gather2_body,
    out_type=jax.ShapeDtypeStruct((_NT, _D), jnp.float32),
    mesh=plsc.VectorSubcoreMesh(core_axis_name="c", subcore_axis_name="s"),
    compiler_params=pltpu.CompilerParams(use_tc_tiling_on_sc=False),
    scratch_types=[
        pltpu.VMEM((_NCHUNK, _CH), jnp.int32),
        pltpu.VMEM((_CH, _D), jnp.float32),
        pltpu.SemaphoreType.DMA,
    ],
)()


def _ema_body(cnt_ref, ec_ref, w_ref, dw_ref, nemb_ref, perp_ref):
    cnt = cnt_ref[:, 0, :]
    ec = ec_ref[...].astype(jnp.float32)
    dc = _DECAY * ec + (1.0 - _DECAY) * cnt
    nsum = jnp.sum(dc, axis=1, keepdims=True)
    nec = (dc + _EPS) / (nsum + _M * _EPS) * nsum
    new_w = _DECAY * w_ref[...] + (1.0 - _DECAY) * dw_ref[...]
    nemb_ref[...] = new_w / nec[:, :, None]
    p = cnt * (1.0 / _T)
    ent = -jnp.sum(p * jnp.log(p + 1e-10), axis=1, keepdims=True)
    perp_ref[...] = jnp.broadcast_to(jnp.sum(jnp.exp(ent)), (1, 1))


def _ema(counts3, ema_count, ema_weight, dw):
    return pl.pallas_call(
        _ema_body,
        out_shape=[
            jax.ShapeDtypeStruct((_N, _M, _D), jnp.float32),
            jax.ShapeDtypeStruct((1, 1), jnp.float32),
        ],
    )(counts3, ema_count, ema_weight, dw)


_LB = 4096
_NLB = _NT // _LB


def _loss_body(x_ref, q_ref, out_ref):
    i = pl.program_id(0)
    d = x_ref[...] - q_ref[...]
    part = jnp.sum(d * d)

    @pl.when(i == 0)
    def _():
        out_ref[...] = jnp.zeros((1, 1), jnp.float32)

    out_ref[...] = out_ref[...] + part

    @pl.when(i == _NLB - 1)
    def _():
        out_ref[...] = out_ref[...] * (_COMMIT / float(_NT * _D))


def _loss(x_rows, q_rows):
    return pl.pallas_call(
        _loss_body,
        grid=(_NLB,),
        in_specs=[
            pl.BlockSpec((_LB, _D), lambda i: (i, 0)),
            pl.BlockSpec((_LB, _D), lambda i: (i, 0)),
        ],
        out_specs=pl.BlockSpec((1, 1), lambda i: (0, 0)),
        out_shape=jax.ShapeDtypeStruct((1, 1), jnp.float32),
        compiler_params=pltpu.CompilerParams(
            dimension_semantics=("arbitrary",)),
    )(x_rows, q_rows)


def kernel(x, embedding, ema_weight, ema_count):
    bs = x.shape[0]
    xr = x.reshape(bs, _N, _D, _L)
    x_flat = jnp.transpose(xr, (1, 0, 3, 2)).reshape(_N, bs * _L, _D)
    e_norm = jnp.sum(embedding ** 2, axis=2)[:, None, :]
    x_norm = jnp.sum(x_flat ** 2, axis=2, keepdims=True)

    idx3, fidx3, counts3 = _dist_argmin(x_flat, embedding, e_norm, x_norm)
    indices = idx3.reshape(_N, _T)
    idx2d = fidx3.reshape(_NT // _CH, _CH)

    emb_flat = embedding.reshape(_NM, _D)
    x_rows = x_flat.reshape(_NT, _D)
    zeros_tile = jnp.zeros((_NM // (_NC * _NS), _D), jnp.float32)
    quant, dw = _sc_gather_scatter(idx2d, x_rows, emb_flat, zeros_tile)

    new_emb, perp2 = _ema(counts3, ema_count, ema_weight, dw.reshape(_N, _M, _D))
    eq_rows = _sc_gather2(idx2d, new_emb.reshape(_NM, _D))
    loss2 = _loss(x_rows, quant)

    loss = loss2.reshape(())
    perplexity = perp2.reshape(())
    z_q = jnp.transpose(quant.reshape(_N, bs, _L, _D), (1, 0, 3, 2)).reshape(bs, _N * _D * _L)
    encodings_q = jnp.transpose(eq_rows.reshape(_N, bs, _L, _D), (1, 0, 3, 2)).reshape(bs, _N * _D, _L, 1)
    indices_out = jnp.transpose(indices.reshape(_N, bs, _L), (1, 0, 2))[..., None]
    return (z_q, loss, perplexity, indices_out, encodings_q)


# speedup vs baseline: 257.7200x; 1.1169x over previous
import functools

import jax
import jax.numpy as jnp
from jax import lax
from jax.experimental import pallas as pl
from jax.experimental.pallas import tpu as pltpu
from jax.experimental.pallas import tpu_sc as plsc

_N = 4
_M = 1024
_D = 64
_L = 16
_B = 1024
_T = _B * _L
_NT = _N * _T
_NM = _N * _M
_DECAY = 0.999
_EPS = 1e-05
_COMMIT = 0.05

_TB = 512
_NTB = _T // _TB

_NC = 2
_NS = 16
_CH = 128
_ROWS_PER_TILE = _NT // (_NC * _NS)
_NCHUNK = _ROWS_PER_TILE // _CH


def _dist_argmin_body(x_ref, emb_ref, en_ref, xn_ref,
                      idx_ref, fidx_ref, cnt_ref, loss_ref):
    n = pl.program_id(0)
    t = pl.program_id(1)
    x = x_ref[0]
    e = emb_ref[0]
    scores = lax.dot_general(
        x, e, (((1,), (1,)), ((), ())),
        preferred_element_type=jnp.float32)
    to_add = en_ref[0] + xn_ref[0]
    dist = to_add - 2.0 * scores
    mn = jnp.min(dist, axis=1, keepdims=True)
    iota_f = lax.broadcasted_iota(jnp.int32, (_TB, _M), 1).astype(jnp.float32)
    idxf = jnp.min(jnp.where(dist == mn, iota_f, float(_M)),
                   axis=1, keepdims=True)
    idx = idxf.astype(jnp.int32)
    idx_ref[...] = idx.reshape(_TB // _L, 1, _L, 1)
    fidx_ref[...] = (idx + n * _M).reshape(1, _TB // _CH, _CH)
    part = jnp.sum((idxf == iota_f).astype(jnp.float32),
                   axis=0, keepdims=True)

    @pl.when(t == 0)
    def _():
        cnt_ref[0] = part

    @pl.when(t != 0)
    def _():
        cnt_ref[0] = cnt_ref[0] + part

    lsum = jnp.sum(mn)

    @pl.when((n == 0) & (t == 0))
    def _():
        loss_ref[...] = jnp.zeros((1, 1), jnp.float32)

    loss_ref[...] = loss_ref[...] + lsum

    @pl.when((n == _N - 1) & (t == _NTB - 1))
    def _():
        loss_ref[...] = loss_ref[...] * (_COMMIT / float(_NT * _D))


def _dist_argmin(x_flat, embedding, e_norm, x_norm):
    out_shapes = [
        jax.ShapeDtypeStruct((_B, _N, _L, 1), jnp.int32),
        jax.ShapeDtypeStruct((_N * _NTB, _TB // _CH, _CH), jnp.int32),
        jax.ShapeDtypeStruct((_N, 1, _M), jnp.float32),
        jax.ShapeDtypeStruct((1, 1), jnp.float32),
    ]
    return pl.pallas_call(
        _dist_argmin_body,
        grid=(_N, _NTB),
        in_specs=[
            pl.BlockSpec((1, _TB, _D), lambda n, t: (n, t, 0)),
            pl.BlockSpec((1, _M, _D), lambda n, t: (n, 0, 0)),
            pl.BlockSpec((1, 1, _M), lambda n, t: (n, 0, 0)),
            pl.BlockSpec((1, _TB, 1), lambda n, t: (n, t, 0)),
        ],
        out_specs=[
            pl.BlockSpec((_TB // _L, 1, _L, 1), lambda n, t: (t, n, 0, 0)),
            pl.BlockSpec((1, _TB // _CH, _CH), lambda n, t: (n * _NTB + t, 0, 0)),
            pl.BlockSpec((1, 1, _M), lambda n, t: (n, 0, 0)),
            pl.BlockSpec((1, 1), lambda n, t: (0, 0)),
        ],
        out_shape=out_shapes,
        compiler_params=pltpu.CompilerParams(
            dimension_semantics=("arbitrary", "arbitrary")),
    )(x_flat, embedding, e_norm, x_norm)


def _sc_gather_scatter_body(idx_hbm, x_hbm, emb_hbm, zero_hbm,
                            q_out, dw_out,
                            idx_v, qrows, xrows, dwsh, sem):
    c = lax.axis_index("c")
    s = lax.axis_index("s")
    gbase = pl.multiple_of(c * (_NM // _NC) + s * (_NM // (_NC * _NS)), 8)
    pltpu.sync_copy(zero_hbm, dwsh.at[pl.ds(gbase, _NM // (_NC * _NS))])
    plsc.subcore_barrier()

    base = c * (_NT // _NC) + s * _ROWS_PER_TILE
    pltpu.sync_copy(idx_hbm.at[pl.ds(pl.multiple_of(base // _CH, 8), _NCHUNK)],
                    idx_v)
    for j in range(_NCHUNK):
        tok = pl.multiple_of(base + j * _CH, 8)
        pltpu.async_copy(emb_hbm.at[idx_v.at[j]], qrows, sem).wait()
        pltpu.sync_copy(qrows, q_out.at[pl.ds(tok, _CH)])
        pltpu.sync_copy(x_hbm.at[pl.ds(tok, _CH)], xrows)
        pltpu.sync_copy(xrows, dwsh.at[idx_v.at[j]], add=True)
    plsc.subcore_barrier()
    pltpu.sync_copy(dwsh.at[pl.ds(gbase, _NM // (_NC * _NS))],
                    dw_out.at[pl.ds(gbase, _NM // (_NC * _NS))])


_sc_gather_scatter = functools.partial(
    pl.kernel,
    _sc_gather_scatter_body,
    out_type=[
        jax.ShapeDtypeStruct((_NT, _D), jnp.float32),
        jax.ShapeDtypeStruct((_NM, _D), jnp.float32),
    ],
    mesh=plsc.VectorSubcoreMesh(core_axis_name="c", subcore_axis_name="s"),
    compiler_params=pltpu.CompilerParams(use_tc_tiling_on_sc=False),
    scratch_types=[
        pltpu.VMEM((_NCHUNK, _CH), jnp.int32),
        pltpu.VMEM((_CH, _D), jnp.float32),
        pltpu.VMEM((_CH, _D), jnp.float32),
        pltpu.VMEM_SHARED((_NM, _D), jnp.float32),
        pltpu.SemaphoreType.DMA,
    ],
)()


def _sc_gather2_body(idx_hbm, emb_hbm, q_out, idx_v, qrows, sem):
    c = lax.axis_index("c")
    s = lax.axis_index("s")
    base = c * (_NT // _NC) + s * _ROWS_PER_TILE
    pltpu.sync_copy(idx_hbm.at[pl.ds(pl.multiple_of(base // _CH, 8), _NCHUNK)],
                    idx_v)
    for j in range(_NCHUNK):
        tok = pl.multiple_of(base + j * _CH, 8)
        pltpu.async_copy(emb_hbm.at[idx_v.at[j]], qrows, sem).wait()
        pltpu.sync_copy(qrows, q_out.at[pl.ds(tok, _CH)])


_sc_gather2 = functools.partial(
    pl.kernel,
    _sc_gather2_body,
    out_type=jax.ShapeDtypeStruct((_NT, _D), jnp.float32),
    mesh=plsc.VectorSubcoreMesh(core_axis_name="c", subcore_axis_name="s"),
    compiler_params=pltpu.CompilerParams(use_tc_tiling_on_sc=False),
    scratch_types=[
        pltpu.VMEM((_NCHUNK, _CH), jnp.int32),
        pltpu.VMEM((_CH, _D), jnp.float32),
        pltpu.SemaphoreType.DMA,
    ],
)()


def _ema_body(cnt_ref, ec_ref, w_ref, dw_ref, nemb_ref, perp_ref):
    cnt = cnt_ref[:, 0, :]
    ec = ec_ref[...].astype(jnp.float32)
    dc = _DECAY * ec + (1.0 - _DECAY) * cnt
    nsum = jnp.sum(dc, axis=1, keepdims=True)
    nec = (dc + _EPS) / (nsum + _M * _EPS) * nsum
    new_w = _DECAY * w_ref[...] + (1.0 - _DECAY) * dw_ref[...]
    nemb_ref[...] = new_w / nec[:, :, None]
    p = cnt * (1.0 / _T)
    ent = -jnp.sum(p * jnp.log(p + 1e-10), axis=1, keepdims=True)
    perp_ref[...] = jnp.broadcast_to(jnp.sum(jnp.exp(ent)), (1, 1))


def _ema(counts3, ema_count, ema_weight, dw):
    return pl.pallas_call(
        _ema_body,
        out_shape=[
            jax.ShapeDtypeStruct((_N, _M, _D), jnp.float32),
            jax.ShapeDtypeStruct((1, 1), jnp.float32),
        ],
    )(counts3, ema_count, ema_weight, dw)


def kernel(x, embedding, ema_weight, ema_count):
    bs = x.shape[0]
    xr = x.reshape(bs, _N, _D, _L)
    x_flat = jnp.transpose(xr, (1, 0, 3, 2)).reshape(_N, bs * _L, _D)
    e_norm = jnp.sum(embedding ** 2, axis=2)[:, None, :]
    x_norm = jnp.sum(x_flat ** 2, axis=2, keepdims=True)

    indices_out, fidx3, counts3, loss2 = _dist_argmin(
        x_flat, embedding, e_norm, x_norm)
    idx2d = fidx3.reshape(_NT // _CH, _CH)

    emb_flat = embedding.reshape(_NM, _D)
    x_rows = x_flat.reshape(_NT, _D)
    zeros_tile = jnp.zeros((_NM // (_NC * _NS), _D), jnp.float32)
    quant, dw = _sc_gather_scatter(idx2d, x_rows, emb_flat, zeros_tile)

    new_emb, perp2 = _ema(counts3, ema_count, ema_weight, dw.reshape(_N, _M, _D))
    eq_rows = _sc_gather2(idx2d, new_emb.reshape(_NM, _D))

    loss = loss2.reshape(())
    perplexity = perp2.reshape(())
    z_q = jnp.transpose(quant.reshape(_N, bs, _L, _D), (1, 0, 3, 2)).reshape(bs, _N * _D * _L)
    encodings_q = jnp.transpose(eq_rows.reshape(_N, bs, _L, _D), (1, 0, 3, 2)).reshape(bs, _N * _D, _L, 1)
    return (z_q, loss, perplexity, indices_out, encodings_q)


# speedup vs baseline: 268.9629x; 1.0436x over previous
import functools

import jax
import jax.numpy as jnp
from jax import lax
from jax.experimental import pallas as pl
from jax.experimental.pallas import tpu as pltpu
from jax.experimental.pallas import tpu_sc as plsc

_N = 4
_M = 1024
_D = 64
_L = 16
_B = 1024
_T = _B * _L
_NT = _N * _T
_NM = _N * _M
_DECAY = 0.999
_EPS = 1e-05
_COMMIT = 0.05

_TB = 512
_NTB = _T // _TB

_NC = 2
_NS = 16
_CH = 128
_ROWS_PER_TILE = _NT // (_NC * _NS)
_NCHUNK = _ROWS_PER_TILE // _CH


def _dist_argmin_body(x_ref, emb_ref, en_ref, xn_ref,
                      idx_ref, fidx_ref, loss_ref):
    n = pl.program_id(0)
    t = pl.program_id(1)
    x = x_ref[0]
    e = emb_ref[0]
    scores = lax.dot_general(
        x, e, (((1,), (1,)), ((), ())),
        preferred_element_type=jnp.float32)
    to_add = en_ref[0] + xn_ref[0]
    dist = to_add - 2.0 * scores
    mn = jnp.min(dist, axis=1, keepdims=True)
    iota_f = lax.broadcasted_iota(jnp.int32, (_TB, _M), 1).astype(jnp.float32)
    idxf = jnp.min(jnp.where(dist == mn, iota_f, float(_M)),
                   axis=1, keepdims=True)
    idx = idxf.astype(jnp.int32)
    idx_ref[...] = idx.reshape(_TB // _L, 1, _L, 1)
    fidx_ref[...] = (idx + n * _M).reshape(1, _TB // _CH, _CH)

    lsum = jnp.sum(mn)

    @pl.when((n == 0) & (t == 0))
    def _():
        loss_ref[...] = jnp.zeros((1, 1), jnp.float32)

    loss_ref[...] = loss_ref[...] + lsum

    @pl.when((n == _N - 1) & (t == _NTB - 1))
    def _():
        loss_ref[...] = loss_ref[...] * (_COMMIT / float(_NT * _D))


def _dist_argmin(x_flat, embedding, e_norm, x_norm):
    out_shapes = [
        jax.ShapeDtypeStruct((_B, _N, _L, 1), jnp.int32),
        jax.ShapeDtypeStruct((_N * _NTB, _TB // _CH, _CH), jnp.int32),
        jax.ShapeDtypeStruct((1, 1), jnp.float32),
    ]
    return pl.pallas_call(
        _dist_argmin_body,
        grid=(_N, _NTB),
        in_specs=[
            pl.BlockSpec((1, _TB, _D), lambda n, t: (n, t, 0)),
            pl.BlockSpec((1, _M, _D), lambda n, t: (n, 0, 0)),
            pl.BlockSpec((1, 1, _M), lambda n, t: (n, 0, 0)),
            pl.BlockSpec((1, _TB, 1), lambda n, t: (n, t, 0)),
        ],
        out_specs=[
            pl.BlockSpec((_TB // _L, 1, _L, 1), lambda n, t: (t, n, 0, 0)),
            pl.BlockSpec((1, _TB // _CH, _CH), lambda n, t: (n * _NTB + t, 0, 0)),
            pl.BlockSpec((1, 1), lambda n, t: (0, 0)),
        ],
        out_shape=out_shapes,
        compiler_params=pltpu.CompilerParams(
            dimension_semantics=("arbitrary", "arbitrary")),
    )(x_flat, embedding, e_norm, x_norm)


def _sc_gather_scatter_body(idx_hbm, x_hbm, emb_hbm, zero_hbm, onesz_hbm,
                            q_out, dw_out, cnt_out,
                            idx_v, qrows, xrows, ones_v, dwsh, csh, sem):
    c = lax.axis_index("c")
    s = lax.axis_index("s")
    gbase = pl.multiple_of(c * (_NM // _NC) + s * (_NM // (_NC * _NS)), 8)
    pltpu.sync_copy(zero_hbm, dwsh.at[pl.ds(gbase, _NM // (_NC * _NS))])
    pltpu.sync_copy(onesz_hbm.at[pl.ds(_CH, _CH)], csh.at[pl.ds(gbase, _NM // (_NC * _NS))])
    pltpu.sync_copy(onesz_hbm.at[pl.ds(0, _CH)], ones_v)
    plsc.subcore_barrier()

    base = c * (_NT // _NC) + s * _ROWS_PER_TILE
    pltpu.sync_copy(idx_hbm.at[pl.ds(pl.multiple_of(base // _CH, 8), _NCHUNK)],
                    idx_v)
    for j in range(_NCHUNK):
        tok = pl.multiple_of(base + j * _CH, 8)
        pltpu.async_copy(emb_hbm.at[idx_v.at[j]], qrows, sem).wait()
        pltpu.sync_copy(qrows, q_out.at[pl.ds(tok, _CH)])
        pltpu.sync_copy(x_hbm.at[pl.ds(tok, _CH)], xrows)
        pltpu.sync_copy(xrows, dwsh.at[idx_v.at[j]], add=True)
        pltpu.sync_copy(ones_v, csh.at[idx_v.at[j]], add=True)
    plsc.subcore_barrier()
    pltpu.sync_copy(dwsh.at[pl.ds(gbase, _NM // (_NC * _NS))],
                    dw_out.at[pl.ds(gbase, _NM // (_NC * _NS))])
    pltpu.sync_copy(csh.at[pl.ds(gbase, _NM // (_NC * _NS))],
                    cnt_out.at[pl.ds(gbase, _NM // (_NC * _NS))])


_sc_gather_scatter = functools.partial(
    pl.kernel,
    _sc_gather_scatter_body,
    out_type=[
        jax.ShapeDtypeStruct((_NT, _D), jnp.float32),
        jax.ShapeDtypeStruct((_NM, _D), jnp.float32),
        jax.ShapeDtypeStruct((_NM, 16), jnp.float32),
    ],
    mesh=plsc.VectorSubcoreMesh(core_axis_name="c", subcore_axis_name="s"),
    compiler_params=pltpu.CompilerParams(use_tc_tiling_on_sc=False),
    scratch_types=[
        pltpu.VMEM((_NCHUNK, _CH), jnp.int32),
        pltpu.VMEM((_CH, _D), jnp.float32),
        pltpu.VMEM((_CH, _D), jnp.float32),
        pltpu.VMEM((_CH, 16), jnp.float32),
        pltpu.VMEM_SHARED((_NM, _D), jnp.float32),
        pltpu.VMEM_SHARED((_NM, 16), jnp.float32),
        pltpu.SemaphoreType.DMA,
    ],
)()


def _sc_gather2_body(idx_hbm, emb_hbm, q_out, idx_v, qrows, sem):
    c = lax.axis_index("c")
    s = lax.axis_index("s")
    base = c * (_NT // _NC) + s * _ROWS_PER_TILE
    pltpu.sync_copy(idx_hbm.at[pl.ds(pl.multiple_of(base // _CH, 8), _NCHUNK)],
                    idx_v)
    for j in range(_NCHUNK):
        tok = pl.multiple_of(base + j * _CH, 8)
        pltpu.async_copy(emb_hbm.at[idx_v.at[j]], qrows, sem).wait()
        pltpu.sync_copy(qrows, q_out.at[pl.ds(tok, _CH)])


_sc_gather2 = functools.partial(
    pl.kernel,
    _sc_gather2_body,
    out_type=jax.ShapeDtypeStruct((_NT, _D), jnp.float32),
    mesh=plsc.VectorSubcoreMesh(core_axis_name="c", subcore_axis_name="s"),
    compiler_params=pltpu.CompilerParams(use_tc_tiling_on_sc=False),
    scratch_types=[
        pltpu.VMEM((_NCHUNK, _CH), jnp.int32),
        pltpu.VMEM((_CH, _D), jnp.float32),
        pltpu.SemaphoreType.DMA,
    ],
)()


def _ema_body(cnt_ref, ec_ref, w_ref, dw_ref, nemb_ref, perp_ref):
    cnt = jnp.sum(cnt_ref[...], axis=2) * (1.0 / 16.0)
    ec = ec_ref[...].astype(jnp.float32)
    dc = _DECAY * ec + (1.0 - _DECAY) * cnt
    nsum = jnp.sum(dc, axis=1, keepdims=True)
    nec = (dc + _EPS) / (nsum + _M * _EPS) * nsum
    new_w = _DECAY * w_ref[...] + (1.0 - _DECAY) * dw_ref[...]
    nemb_ref[...] = new_w / nec[:, :, None]
    p = cnt * (1.0 / _T)
    ent = -jnp.sum(p * jnp.log(p + 1e-10), axis=1, keepdims=True)
    perp_ref[...] = jnp.broadcast_to(jnp.sum(jnp.exp(ent)), (1, 1))


def _ema(counts3, ema_count, ema_weight, dw):
    return pl.pallas_call(
        _ema_body,
        out_shape=[
            jax.ShapeDtypeStruct((_N, _M, _D), jnp.float32),
            jax.ShapeDtypeStruct((1, 1), jnp.float32),
        ],
    )(counts3, ema_count, ema_weight, dw)


def kernel(x, embedding, ema_weight, ema_count):
    bs = x.shape[0]
    xr = x.reshape(bs, _N, _D, _L)
    x_flat = jnp.transpose(xr, (1, 0, 3, 2)).reshape(_N, bs * _L, _D)
    e_norm = jnp.sum(embedding ** 2, axis=2)[:, None, :]
    x_norm = jnp.sum(x_flat ** 2, axis=2, keepdims=True)

    indices_out, fidx3, loss2 = _dist_argmin(
        x_flat, embedding, e_norm, x_norm)
    idx2d = fidx3.reshape(_NT // _CH, _CH)

    emb_flat = embedding.reshape(_NM, _D)
    x_rows = x_flat.reshape(_NT, _D)
    zeros_tile = jnp.zeros((_NM // (_NC * _NS), _D), jnp.float32)
    onesz = jnp.concatenate([jnp.ones((_CH, 16), jnp.float32),
                             jnp.zeros((_CH, 16), jnp.float32)], axis=0)
    quant, dw, cnt16 = _sc_gather_scatter(idx2d, x_rows, emb_flat, zeros_tile,
                                          onesz)

    new_emb, perp2 = _ema(cnt16.reshape(_N, _M, 16), ema_count, ema_weight,
                          dw.reshape(_N, _M, _D))
    eq_rows = _sc_gather2(idx2d, new_emb.reshape(_NM, _D))

    loss = loss2.reshape(())
    perplexity = perp2.reshape(())
    z_q = jnp.transpose(quant.reshape(_N, bs, _L, _D), (1, 0, 3, 2)).reshape(bs, _N * _D * _L)
    encodings_q = jnp.transpose(eq_rows.reshape(_N, bs, _L, _D), (1, 0, 3, 2)).reshape(bs, _N * _D, _L, 1)
    return (z_q, loss, perplexity, indices_out, encodings_q)


# speedup vs baseline: 292.5602x; 1.0877x over previous
import functools

import jax
import jax.numpy as jnp
from jax import lax
from jax.experimental import pallas as pl
from jax.experimental.pallas import tpu as pltpu
from jax.experimental.pallas import tpu_sc as plsc

_N = 4
_M = 1024
_D = 64
_L = 16
_B = 1024
_T = _B * _L
_NT = _N * _T
_NM = _N * _M
_DECAY = 0.999
_EPS = 1e-05
_COMMIT = 0.05

_TB = 1024
_NTB = _T // _TB

_NC = 2
_NS = 16
_CH = 128
_ROWS_PER_TILE = _NT // (_NC * _NS)
_NCHUNK = _ROWS_PER_TILE // _CH


def _dist_argmin_body(x_ref, emb_ref, en_ref, xn_ref,
                      idx_ref, fidx_ref, loss_ref):
    n = pl.program_id(0)
    t = pl.program_id(1)
    x = x_ref[0]
    e = emb_ref[0]
    scores = lax.dot_general(
        x, e, (((1,), (1,)), ((), ())),
        preferred_element_type=jnp.float32)
    to_add = en_ref[0] + xn_ref[0]
    dist = to_add - 2.0 * scores
    mn = jnp.min(dist, axis=1, keepdims=True)
    iota_f = lax.broadcasted_iota(jnp.int32, (_TB, _M), 1).astype(jnp.float32)
    idxf = jnp.min(jnp.where(dist == mn, iota_f, float(_M)),
                   axis=1, keepdims=True)
    idx = idxf.astype(jnp.int32)
    idx_ref[...] = idx.reshape(_TB // _L, 1, _L, 1)
    fidx_ref[...] = (idx + n * _M).reshape(1, _TB // _CH, _CH)

    lsum = jnp.sum(mn)

    @pl.when((n == 0) & (t == 0))
    def _():
        loss_ref[...] = jnp.zeros((1, 1), jnp.float32)

    loss_ref[...] = loss_ref[...] + lsum

    @pl.when((n == _N - 1) & (t == _NTB - 1))
    def _():
        loss_ref[...] = loss_ref[...] * (_COMMIT / float(_NT * _D))


def _dist_argmin(x_flat, embedding, e_norm, x_norm):
    out_shapes = [
        jax.ShapeDtypeStruct((_B, _N, _L, 1), jnp.int32),
        jax.ShapeDtypeStruct((_N * _NTB, _TB // _CH, _CH), jnp.int32),
        jax.ShapeDtypeStruct((1, 1), jnp.float32),
    ]
    return pl.pallas_call(
        _dist_argmin_body,
        grid=(_N, _NTB),
        in_specs=[
            pl.BlockSpec((1, _TB, _D), lambda n, t: (n, t, 0)),
            pl.BlockSpec((1, _M, _D), lambda n, t: (n, 0, 0)),
            pl.BlockSpec((1, 1, _M), lambda n, t: (n, 0, 0)),
            pl.BlockSpec((1, _TB, 1), lambda n, t: (n, t, 0)),
        ],
        out_specs=[
            pl.BlockSpec((_TB // _L, 1, _L, 1), lambda n, t: (t, n, 0, 0)),
            pl.BlockSpec((1, _TB // _CH, _CH), lambda n, t: (n * _NTB + t, 0, 0)),
            pl.BlockSpec((1, 1), lambda n, t: (0, 0)),
        ],
        out_shape=out_shapes,
        compiler_params=pltpu.CompilerParams(
            dimension_semantics=("arbitrary", "arbitrary")),
    )(x_flat, embedding, e_norm, x_norm)


def _sc_gather_scatter_body(idx_hbm, x_hbm, emb_hbm, zero_hbm, onesz_hbm,
                            q_out, dw_out, cnt_out,
                            idx_v, qrows, xrows, ones_v, dwsh, csh, sem):
    c = lax.axis_index("c")
    s = lax.axis_index("s")
    gbase = pl.multiple_of(c * (_NM // _NC) + s * (_NM // (_NC * _NS)), 8)
    pltpu.sync_copy(zero_hbm, dwsh.at[pl.ds(gbase, _NM // (_NC * _NS))])
    pltpu.sync_copy(onesz_hbm.at[pl.ds(_CH, _CH)], csh.at[pl.ds(gbase, _NM // (_NC * _NS))])
    pltpu.sync_copy(onesz_hbm.at[pl.ds(0, _CH)], ones_v)
    plsc.subcore_barrier()

    base = c * (_NT // _NC) + s * _ROWS_PER_TILE
    pltpu.sync_copy(idx_hbm.at[pl.ds(pl.multiple_of(base // _CH, 8), _NCHUNK)],
                    idx_v)
    for j in range(_NCHUNK):
        tok = pl.multiple_of(base + j * _CH, 8)
        pltpu.async_copy(emb_hbm.at[idx_v.at[j]], qrows, sem).wait()
        pltpu.sync_copy(qrows, q_out.at[pl.ds(tok, _CH)])
        pltpu.sync_copy(x_hbm.at[pl.ds(tok, _CH)], xrows)
        pltpu.sync_copy(xrows, dwsh.at[idx_v.at[j]], add=True)
        pltpu.sync_copy(ones_v, csh.at[idx_v.at[j]], add=True)
    plsc.subcore_barrier()
    pltpu.sync_copy(dwsh.at[pl.ds(gbase, _NM // (_NC * _NS))],
                    dw_out.at[pl.ds(gbase, _NM // (_NC * _NS))])
    pltpu.sync_copy(csh.at[pl.ds(gbase, _NM // (_NC * _NS))],
                    cnt_out.at[pl.ds(gbase, _NM // (_NC * _NS))])


_sc_gather_scatter = functools.partial(
    pl.kernel,
    _sc_gather_scatter_body,
    out_type=[
        jax.ShapeDtypeStruct((_NT, _D), jnp.float32),
        jax.ShapeDtypeStruct((_NM, _D), jnp.float32),
        jax.ShapeDtypeStruct((_NM, 16), jnp.float32),
    ],
    mesh=plsc.VectorSubcoreMesh(core_axis_name="c", subcore_axis_name="s"),
    compiler_params=pltpu.CompilerParams(use_tc_tiling_on_sc=False),
    scratch_types=[
        pltpu.VMEM((_NCHUNK, _CH), jnp.int32),
        pltpu.VMEM((_CH, _D), jnp.float32),
        pltpu.VMEM((_CH, _D), jnp.float32),
        pltpu.VMEM((_CH, 16), jnp.float32),
        pltpu.VMEM_SHARED((_NM, _D), jnp.float32),
        pltpu.VMEM_SHARED((_NM, 16), jnp.float32),
        pltpu.SemaphoreType.DMA,
    ],
)()


def _sc_gather2_body(idx_hbm, emb_hbm, q_out, idx_v, qrows, sem):
    c = lax.axis_index("c")
    s = lax.axis_index("s")
    base = c * (_NT // _NC) + s * _ROWS_PER_TILE
    pltpu.sync_copy(idx_hbm.at[pl.ds(pl.multiple_of(base // _CH, 8), _NCHUNK)],
                    idx_v)
    for j in range(_NCHUNK):
        tok = pl.multiple_of(base + j * _CH, 8)
        pltpu.async_copy(emb_hbm.at[idx_v.at[j]], qrows, sem).wait()
        pltpu.sync_copy(qrows, q_out.at[pl.ds(tok, _CH)])


_sc_gather2 = functools.partial(
    pl.kernel,
    _sc_gather2_body,
    out_type=jax.ShapeDtypeStruct((_NT, _D), jnp.float32),
    mesh=plsc.VectorSubcoreMesh(core_axis_name="c", subcore_axis_name="s"),
    compiler_params=pltpu.CompilerParams(use_tc_tiling_on_sc=False),
    scratch_types=[
        pltpu.VMEM((_NCHUNK, _CH), jnp.int32),
        pltpu.VMEM((_CH, _D), jnp.float32),
        pltpu.SemaphoreType.DMA,
    ],
)()


def _ema_body(cnt_ref, ec_ref, w_ref, dw_ref, nemb_ref, perp_ref):
    cnt = jnp.sum(cnt_ref[...], axis=2) * (1.0 / 16.0)
    ec = ec_ref[...].astype(jnp.float32)
    dc = _DECAY * ec + (1.0 - _DECAY) * cnt
    nsum = jnp.sum(dc, axis=1, keepdims=True)
    nec = (dc + _EPS) / (nsum + _M * _EPS) * nsum
    new_w = _DECAY * w_ref[...] + (1.0 - _DECAY) * dw_ref[...]
    nemb_ref[...] = new_w / nec[:, :, None]
    p = cnt * (1.0 / _T)
    ent = -jnp.sum(p * jnp.log(p + 1e-10), axis=1, keepdims=True)
    perp_ref[...] = jnp.broadcast_to(jnp.sum(jnp.exp(ent)), (1, 1))


def _ema(counts3, ema_count, ema_weight, dw):
    return pl.pallas_call(
        _ema_body,
        out_shape=[
            jax.ShapeDtypeStruct((_N, _M, _D), jnp.float32),
            jax.ShapeDtypeStruct((1, 1), jnp.float32),
        ],
    )(counts3, ema_count, ema_weight, dw)


def kernel(x, embedding, ema_weight, ema_count):
    bs = x.shape[0]
    xr = x.reshape(bs, _N, _D, _L)
    x_flat = jnp.transpose(xr, (1, 0, 3, 2)).reshape(_N, bs * _L, _D)
    e_norm = jnp.sum(embedding ** 2, axis=2)[:, None, :]
    x_norm = jnp.sum(x_flat ** 2, axis=2, keepdims=True)

    indices_out, fidx3, loss2 = _dist_argmin(
        x_flat, embedding, e_norm, x_norm)
    idx2d = fidx3.reshape(_NT // _CH, _CH)

    emb_flat = embedding.reshape(_NM, _D)
    x_rows = x_flat.reshape(_NT, _D)
    zeros_tile = jnp.zeros((_NM // (_NC * _NS), _D), jnp.float32)
    onesz = jnp.concatenate([jnp.ones((_CH, 16), jnp.float32),
                             jnp.zeros((_CH, 16), jnp.float32)], axis=0)
    quant, dw, cnt16 = _sc_gather_scatter(idx2d, x_rows, emb_flat, zeros_tile,
                                          onesz)

    new_emb, perp2 = _ema(cnt16.reshape(_N, _M, 16), ema_count, ema_weight,
                          dw.reshape(_N, _M, _D))
    eq_rows = _sc_gather2(idx2d, new_emb.reshape(_NM, _D))

    loss = loss2.reshape(())
    perplexity = perp2.reshape(())
    z_q = jnp.transpose(quant.reshape(_N, bs, _L, _D), (1, 0, 3, 2)).reshape(bs, _N * _D * _L)
    encodings_q = jnp.transpose(eq_rows.reshape(_N, bs, _L, _D), (1, 0, 3, 2)).reshape(bs, _N * _D, _L, 1)
    return (z_q, loss, perplexity, indices_out, encodings_q)


# speedup vs baseline: 301.4844x; 1.0305x over previous
import functools

import jax
import jax.numpy as jnp
from jax import lax
from jax.experimental import pallas as pl
from jax.experimental.pallas import tpu as pltpu
from jax.experimental.pallas import tpu_sc as plsc

_N = 4
_M = 1024
_D = 64
_L = 16
_B = 1024
_T = _B * _L
_NT = _N * _T
_NM = _N * _M
_DECAY = 0.999
_EPS = 1e-05
_COMMIT = 0.05

_TB = 2048
_NTB = _T // _TB

_NC = 2
_NS = 16
_CH = 128
_ROWS_PER_TILE = _NT // (_NC * _NS)
_NCHUNK = _ROWS_PER_TILE // _CH


def _dist_argmin_body(x_ref, emb_ref, en_ref, xn_ref,
                      idx_ref, fidx_ref, loss_ref):
    n = pl.program_id(0)
    t = pl.program_id(1)
    x = x_ref[0]
    e = emb_ref[0]
    scores = lax.dot_general(
        x, e, (((1,), (1,)), ((), ())),
        preferred_element_type=jnp.float32)
    to_add = en_ref[0] + xn_ref[0]
    dist = to_add - 2.0 * scores
    mn = jnp.min(dist, axis=1, keepdims=True)
    iota_f = lax.broadcasted_iota(jnp.int32, (_TB, _M), 1).astype(jnp.float32)
    idxf = jnp.min(jnp.where(dist == mn, iota_f, float(_M)),
                   axis=1, keepdims=True)
    idx = idxf.astype(jnp.int32)
    idx_ref[...] = idx.reshape(_TB // _L, 1, _L, 1)
    fidx_ref[...] = (idx + n * _M).reshape(1, _TB // _CH, _CH)

    lsum = jnp.sum(mn)

    @pl.when((n == 0) & (t == 0))
    def _():
        loss_ref[...] = jnp.zeros((1, 1), jnp.float32)

    loss_ref[...] = loss_ref[...] + lsum

    @pl.when((n == _N - 1) & (t == _NTB - 1))
    def _():
        loss_ref[...] = loss_ref[...] * (_COMMIT / float(_NT * _D))


def _dist_argmin(x_flat, embedding, e_norm, x_norm):
    out_shapes = [
        jax.ShapeDtypeStruct((_B, _N, _L, 1), jnp.int32),
        jax.ShapeDtypeStruct((_N * _NTB, _TB // _CH, _CH), jnp.int32),
        jax.ShapeDtypeStruct((1, 1), jnp.float32),
    ]
    return pl.pallas_call(
        _dist_argmin_body,
        grid=(_N, _NTB),
        in_specs=[
            pl.BlockSpec((1, _TB, _D), lambda n, t: (n, t, 0)),
            pl.BlockSpec((1, _M, _D), lambda n, t: (n, 0, 0)),
            pl.BlockSpec((1, 1, _M), lambda n, t: (n, 0, 0)),
            pl.BlockSpec((1, _TB, 1), lambda n, t: (n, t, 0)),
        ],
        out_specs=[
            pl.BlockSpec((_TB // _L, 1, _L, 1), lambda n, t: (t, n, 0, 0)),
            pl.BlockSpec((1, _TB // _CH, _CH), lambda n, t: (n * _NTB + t, 0, 0)),
            pl.BlockSpec((1, 1), lambda n, t: (0, 0)),
        ],
        out_shape=out_shapes,
        compiler_params=pltpu.CompilerParams(
            dimension_semantics=("arbitrary", "arbitrary")),
    )(x_flat, embedding, e_norm, x_norm)


def _sc_gather_scatter_body(idx_hbm, x_hbm, emb_hbm, zero_hbm, onesz_hbm,
                            q_out, dw_out, cnt_out,
                            idx_v, qrows, xrows, ones_v, dwsh, csh, sem):
    c = lax.axis_index("c")
    s = lax.axis_index("s")
    gbase = pl.multiple_of(c * (_NM // _NC) + s * (_NM // (_NC * _NS)), 8)
    pltpu.sync_copy(zero_hbm, dwsh.at[pl.ds(gbase, _NM // (_NC * _NS))])
    pltpu.sync_copy(onesz_hbm.at[pl.ds(_CH, _CH)], csh.at[pl.ds(gbase, _NM // (_NC * _NS))])
    pltpu.sync_copy(onesz_hbm.at[pl.ds(0, _CH)], ones_v)
    plsc.subcore_barrier()

    base = c * (_NT // _NC) + s * _ROWS_PER_TILE
    pltpu.sync_copy(idx_hbm.at[pl.ds(pl.multiple_of(base // _CH, 8), _NCHUNK)],
                    idx_v)
    for j in range(_NCHUNK):
        tok = pl.multiple_of(base + j * _CH, 8)
        pltpu.async_copy(emb_hbm.at[idx_v.at[j]], qrows, sem).wait()
        pltpu.sync_copy(qrows, q_out.at[pl.ds(tok, _CH)])
        pltpu.sync_copy(x_hbm.at[pl.ds(tok, _CH)], xrows)
        pltpu.sync_copy(xrows, dwsh.at[idx_v.at[j]], add=True)
        pltpu.sync_copy(ones_v, csh.at[idx_v.at[j]], add=True)
    plsc.subcore_barrier()
    pltpu.sync_copy(dwsh.at[pl.ds(gbase, _NM // (_NC * _NS))],
                    dw_out.at[pl.ds(gbase, _NM // (_NC * _NS))])
    pltpu.sync_copy(csh.at[pl.ds(gbase, _NM // (_NC * _NS))],
                    cnt_out.at[pl.ds(gbase, _NM // (_NC * _NS))])


_sc_gather_scatter = functools.partial(
    pl.kernel,
    _sc_gather_scatter_body,
    out_type=[
        jax.ShapeDtypeStruct((_NT, _D), jnp.float32),
        jax.ShapeDtypeStruct((_NM, _D), jnp.float32),
        jax.ShapeDtypeStruct((_NM, 16), jnp.float32),
    ],
    mesh=plsc.VectorSubcoreMesh(core_axis_name="c", subcore_axis_name="s"),
    compiler_params=pltpu.CompilerParams(use_tc_tiling_on_sc=False),
    scratch_types=[
        pltpu.VMEM((_NCHUNK, _CH), jnp.int32),
        pltpu.VMEM((_CH, _D), jnp.float32),
        pltpu.VMEM((_CH, _D), jnp.float32),
        pltpu.VMEM((_CH, 16), jnp.float32),
        pltpu.VMEM_SHARED((_NM, _D), jnp.float32),
        pltpu.VMEM_SHARED((_NM, 16), jnp.float32),
        pltpu.SemaphoreType.DMA,
    ],
)()


def _sc_gather2_body(idx_hbm, emb_hbm, q_out, idx_v, qrows, sem):
    c = lax.axis_index("c")
    s = lax.axis_index("s")
    base = c * (_NT // _NC) + s * _ROWS_PER_TILE
    pltpu.sync_copy(idx_hbm.at[pl.ds(pl.multiple_of(base // _CH, 8), _NCHUNK)],
                    idx_v)
    for j in range(_NCHUNK):
        tok = pl.multiple_of(base + j * _CH, 8)
        pltpu.async_copy(emb_hbm.at[idx_v.at[j]], qrows, sem).wait()
        pltpu.sync_copy(qrows, q_out.at[pl.ds(tok, _CH)])


_sc_gather2 = functools.partial(
    pl.kernel,
    _sc_gather2_body,
    out_type=jax.ShapeDtypeStruct((_NT, _D), jnp.float32),
    mesh=plsc.VectorSubcoreMesh(core_axis_name="c", subcore_axis_name="s"),
    compiler_params=pltpu.CompilerParams(use_tc_tiling_on_sc=False),
    scratch_types=[
        pltpu.VMEM((_NCHUNK, _CH), jnp.int32),
        pltpu.VMEM((_CH, _D), jnp.float32),
        pltpu.SemaphoreType.DMA,
    ],
)()


def _ema_body(cnt_ref, ec_ref, w_ref, dw_ref, nemb_ref, perp_ref):
    cnt = jnp.sum(cnt_ref[...], axis=2) * (1.0 / 16.0)
    ec = ec_ref[...].astype(jnp.float32)
    dc = _DECAY * ec + (1.0 - _DECAY) * cnt
    nsum = jnp.sum(dc, axis=1, keepdims=True)
    nec = (dc + _EPS) / (nsum + _M * _EPS) * nsum
    new_w = _DECAY * w_ref[...] + (1.0 - _DECAY) * dw_ref[...]
    nemb_ref[...] = new_w / nec[:, :, None]
    p = cnt * (1.0 / _T)
    ent = -jnp.sum(p * jnp.log(p + 1e-10), axis=1, keepdims=True)
    perp_ref[...] = jnp.broadcast_to(jnp.sum(jnp.exp(ent)), (1, 1))


def _ema(counts3, ema_count, ema_weight, dw):
    return pl.pallas_call(
        _ema_body,
        out_shape=[
            jax.ShapeDtypeStruct((_N, _M, _D), jnp.float32),
            jax.ShapeDtypeStruct((1, 1), jnp.float32),
        ],
    )(counts3, ema_count, ema_weight, dw)


def kernel(x, embedding, ema_weight, ema_count):
    bs = x.shape[0]
    xr = x.reshape(bs, _N, _D, _L)
    x_flat = jnp.transpose(xr, (1, 0, 3, 2)).reshape(_N, bs * _L, _D)
    e_norm = jnp.sum(embedding ** 2, axis=2)[:, None, :]
    x_norm = jnp.sum(x_flat ** 2, axis=2, keepdims=True)

    indices_out, fidx3, loss2 = _dist_argmin(
        x_flat, embedding, e_norm, x_norm)
    idx2d = fidx3.reshape(_NT // _CH, _CH)

    emb_flat = embedding.reshape(_NM, _D)
    x_rows = x_flat.reshape(_NT, _D)
    zeros_tile = jnp.zeros((_NM // (_NC * _NS), _D), jnp.float32)
    onesz = jnp.concatenate([jnp.ones((_CH, 16), jnp.float32),
                             jnp.zeros((_CH, 16), jnp.float32)], axis=0)
    quant, dw, cnt16 = _sc_gather_scatter(idx2d, x_rows, emb_flat, zeros_tile,
                                          onesz)

    new_emb, perp2 = _ema(cnt16.reshape(_N, _M, 16), ema_count, ema_weight,
                          dw.reshape(_N, _M, _D))
    eq_rows = _sc_gather2(idx2d, new_emb.reshape(_NM, _D))

    loss = loss2.reshape(())
    perplexity = perp2.reshape(())
    z_q = jnp.transpose(quant.reshape(_N, bs, _L, _D), (1, 0, 3, 2)).reshape(bs, _N * _D * _L)
    encodings_q = jnp.transpose(eq_rows.reshape(_N, bs, _L, _D), (1, 0, 3, 2)).reshape(bs, _N * _D, _L, 1)
    return (z_q, loss, perplexity, indices_out, encodings_q)


# speedup vs baseline: 327.0121x; 1.0847x over previous
import functools

import jax
import jax.numpy as jnp
from jax import lax
from jax.experimental import pallas as pl
from jax.experimental.pallas import tpu as pltpu
from jax.experimental.pallas import tpu_sc as plsc

_N = 4
_M = 1024
_D = 64
_L = 16
_B = 1024
_T = _B * _L
_NT = _N * _T
_NM = _N * _M
_DECAY = 0.999
_EPS = 1e-05
_COMMIT = 0.05

_TB = 2048
_NTB = _T // _TB

_NC = 2
_NS = 16
_CH = 128
_ROWS_PER_TILE = _NT // (_NC * _NS)
_NCHUNK = _ROWS_PER_TILE // _CH


def _dist_argmin_body(x_ref, emb_ref, en_ref,
                      idx_ref, fidx_ref, loss_ref):
    n = pl.program_id(0)
    t = pl.program_id(1)
    x = x_ref[0]
    e = emb_ref[0]
    scores = lax.dot_general(
        x, e, (((1,), (1,)), ((), ())),
        preferred_element_type=jnp.float32)
    xn = jnp.sum(x * x, axis=1, keepdims=True)
    to_add = en_ref[0] + xn
    dist = to_add - 2.0 * scores
    mn = jnp.min(dist, axis=1, keepdims=True)
    iota_f = lax.broadcasted_iota(jnp.int32, (_TB, _M), 1).astype(jnp.float32)
    idxf = jnp.min(jnp.where(dist == mn, iota_f, float(_M)),
                   axis=1, keepdims=True)
    idx = idxf.astype(jnp.int32)
    idx_ref[...] = idx.reshape(_TB // _L, 1, _L, 1)
    fidx_ref[...] = (idx + n * _M).reshape(1, _TB // _CH, _CH)

    lsum = jnp.sum(mn)

    @pl.when((n == 0) & (t == 0))
    def _():
        loss_ref[...] = jnp.zeros((1, 1), jnp.float32)

    loss_ref[...] = loss_ref[...] + lsum

    @pl.when((n == _N - 1) & (t == _NTB - 1))
    def _():
        loss_ref[...] = loss_ref[...] * (_COMMIT / float(_NT * _D))


def _dist_argmin(x_flat, embedding, e_norm):
    out_shapes = [
        jax.ShapeDtypeStruct((_B, _N, _L, 1), jnp.int32),
        jax.ShapeDtypeStruct((_N * _NTB, _TB // _CH, _CH), jnp.int32),
        jax.ShapeDtypeStruct((1, 1), jnp.float32),
    ]
    return pl.pallas_call(
        _dist_argmin_body,
        grid=(_N, _NTB),
        in_specs=[
            pl.BlockSpec((1, _TB, _D), lambda n, t: (n, t, 0)),
            pl.BlockSpec((1, _M, _D), lambda n, t: (n, 0, 0)),
            pl.BlockSpec((1, 1, _M), lambda n, t: (n, 0, 0)),
        ],
        out_specs=[
            pl.BlockSpec((_TB // _L, 1, _L, 1), lambda n, t: (t, n, 0, 0)),
            pl.BlockSpec((1, _TB // _CH, _CH), lambda n, t: (n * _NTB + t, 0, 0)),
            pl.BlockSpec((1, 1), lambda n, t: (0, 0)),
        ],
        out_shape=out_shapes,
        compiler_params=pltpu.CompilerParams(
            dimension_semantics=("arbitrary", "arbitrary")),
    )(x_flat, embedding, e_norm)


def _sc_gather_scatter_body(idx_hbm, x_hbm, emb_hbm, zero_hbm, onesz_hbm,
                            q_out, dw_out, cnt_out,
                            idx_v, qrows, xrows, ones_v, dwsh, csh, sem):
    c = lax.axis_index("c")
    s = lax.axis_index("s")
    gbase = pl.multiple_of(c * (_NM // _NC) + s * (_NM // (_NC * _NS)), 8)
    pltpu.sync_copy(zero_hbm, dwsh.at[pl.ds(gbase, _NM // (_NC * _NS))])
    pltpu.sync_copy(onesz_hbm.at[pl.ds(_CH, _CH)], csh.at[pl.ds(gbase, _NM // (_NC * _NS))])
    pltpu.sync_copy(onesz_hbm.at[pl.ds(0, _CH)], ones_v)
    plsc.subcore_barrier()

    base = c * (_NT // _NC) + s * _ROWS_PER_TILE
    pltpu.sync_copy(idx_hbm.at[pl.ds(pl.multiple_of(base // _CH, 8), _NCHUNK)],
                    idx_v)
    for j in range(_NCHUNK):
        tok = pl.multiple_of(base + j * _CH, 8)
        pltpu.async_copy(emb_hbm.at[idx_v.at[j]], qrows, sem).wait()
        pltpu.sync_copy(qrows, q_out.at[pl.ds(tok, _CH)])
        pltpu.sync_copy(x_hbm.at[pl.ds(tok, _CH)], xrows)
        pltpu.sync_copy(xrows, dwsh.at[idx_v.at[j]], add=True)
        pltpu.sync_copy(ones_v, csh.at[idx_v.at[j]], add=True)
    plsc.subcore_barrier()
    pltpu.sync_copy(dwsh.at[pl.ds(gbase, _NM // (_NC * _NS))],
                    dw_out.at[pl.ds(gbase, _NM // (_NC * _NS))])
    pltpu.sync_copy(csh.at[pl.ds(gbase, _NM // (_NC * _NS))],
                    cnt_out.at[pl.ds(gbase, _NM // (_NC * _NS))])


_sc_gather_scatter = functools.partial(
    pl.kernel,
    _sc_gather_scatter_body,
    out_type=[
        jax.ShapeDtypeStruct((_NT, _D), jnp.float32),
        jax.ShapeDtypeStruct((_NM, _D), jnp.float32),
        jax.ShapeDtypeStruct((_NM, 16), jnp.float32),
    ],
    mesh=plsc.VectorSubcoreMesh(core_axis_name="c", subcore_axis_name="s"),
    compiler_params=pltpu.CompilerParams(use_tc_tiling_on_sc=False),
    scratch_types=[
        pltpu.VMEM((_NCHUNK, _CH), jnp.int32),
        pltpu.VMEM((_CH, _D), jnp.float32),
        pltpu.VMEM((_CH, _D), jnp.float32),
        pltpu.VMEM((_CH, 16), jnp.float32),
        pltpu.VMEM_SHARED((_NM, _D), jnp.float32),
        pltpu.VMEM_SHARED((_NM, 16), jnp.float32),
        pltpu.SemaphoreType.DMA,
    ],
)()


def _sc_gather2_body(idx_hbm, emb_hbm, q_out, idx_v, qrows, sem):
    c = lax.axis_index("c")
    s = lax.axis_index("s")
    base = c * (_NT // _NC) + s * _ROWS_PER_TILE
    pltpu.sync_copy(idx_hbm.at[pl.ds(pl.multiple_of(base // _CH, 8), _NCHUNK)],
                    idx_v)
    for j in range(_NCHUNK):
        tok = pl.multiple_of(base + j * _CH, 8)
        pltpu.async_copy(emb_hbm.at[idx_v.at[j]], qrows, sem).wait()
        pltpu.sync_copy(qrows, q_out.at[pl.ds(tok, _CH)])


_sc_gather2 = functools.partial(
    pl.kernel,
    _sc_gather2_body,
    out_type=jax.ShapeDtypeStruct((_NT, _D), jnp.float32),
    mesh=plsc.VectorSubcoreMesh(core_axis_name="c", subcore_axis_name="s"),
    compiler_params=pltpu.CompilerParams(use_tc_tiling_on_sc=False),
    scratch_types=[
        pltpu.VMEM((_NCHUNK, _CH), jnp.int32),
        pltpu.VMEM((_CH, _D), jnp.float32),
        pltpu.SemaphoreType.DMA,
    ],
)()


def _ema_body(cnt_ref, ec_ref, w_ref, dw_ref, nemb_ref, perp_ref):
    cnt = jnp.sum(cnt_ref[...], axis=2) * (1.0 / 16.0)
    ec = ec_ref[...].astype(jnp.float32)
    dc = _DECAY * ec + (1.0 - _DECAY) * cnt
    nsum = jnp.sum(dc, axis=1, keepdims=True)
    nec = (dc + _EPS) / (nsum + _M * _EPS) * nsum
    new_w = _DECAY * w_ref[...] + (1.0 - _DECAY) * dw_ref[...]
    nemb_ref[...] = new_w / nec[:, :, None]
    p = cnt * (1.0 / _T)
    ent = -jnp.sum(p * jnp.log(p + 1e-10), axis=1, keepdims=True)
    perp_ref[...] = jnp.broadcast_to(jnp.sum(jnp.exp(ent)), (1, 1))


def _ema(counts3, ema_count, ema_weight, dw):
    return pl.pallas_call(
        _ema_body,
        out_shape=[
            jax.ShapeDtypeStruct((_N, _M, _D), jnp.float32),
            jax.ShapeDtypeStruct((1, 1), jnp.float32),
        ],
    )(counts3, ema_count, ema_weight, dw)


def kernel(x, embedding, ema_weight, ema_count):
    bs = x.shape[0]
    xr = x.reshape(bs, _N, _D, _L)
    x_flat = jnp.transpose(xr, (1, 0, 3, 2)).reshape(_N, bs * _L, _D)
    e_norm = jnp.sum(embedding ** 2, axis=2)[:, None, :]

    indices_out, fidx3, loss2 = _dist_argmin(x_flat, embedding, e_norm)
    idx2d = fidx3.reshape(_NT // _CH, _CH)

    emb_flat = embedding.reshape(_NM, _D)
    x_rows = x_flat.reshape(_NT, _D)
    zeros_tile = jnp.zeros((_NM // (_NC * _NS), _D), jnp.float32)
    onesz = jnp.concatenate([jnp.ones((_CH, 16), jnp.float32),
                             jnp.zeros((_CH, 16), jnp.float32)], axis=0)
    quant, dw, cnt16 = _sc_gather_scatter(idx2d, x_rows, emb_flat, zeros_tile,
                                          onesz)

    new_emb, perp2 = _ema(cnt16.reshape(_N, _M, 16), ema_count, ema_weight,
                          dw.reshape(_N, _M, _D))
    eq_rows = _sc_gather2(idx2d, new_emb.reshape(_NM, _D))

    loss = loss2.reshape(())
    perplexity = perp2.reshape(())
    z_q = jnp.transpose(quant.reshape(_N, bs, _L, _D), (1, 0, 3, 2)).reshape(bs, _N * _D * _L)
    encodings_q = jnp.transpose(eq_rows.reshape(_N, bs, _L, _D), (1, 0, 3, 2)).reshape(bs, _N * _D, _L, 1)
    return (z_q, loss, perplexity, indices_out, encodings_q)


# speedup vs baseline: 353.3413x; 1.0805x over previous
import functools

import jax
import jax.numpy as jnp
from jax import lax
from jax.experimental import pallas as pl
from jax.experimental.pallas import tpu as pltpu
from jax.experimental.pallas import tpu_sc as plsc

_N = 4
_M = 1024
_D = 64
_L = 16
_B = 1024
_T = _B * _L
_NT = _N * _T
_NM = _N * _M
_DECAY = 0.999
_EPS = 1e-05
_COMMIT = 0.05

_TB = 2048
_NTB = _T // _TB

_NC = 2
_NS = 16
_CH = 128
_ROWS_PER_TILE = _NT // (_NC * _NS)
_NCHUNK = _ROWS_PER_TILE // _CH


def _dist_argmin_body(x_ref, emb_ref, en_ref,
                      idx_ref, fidx_ref, loss_ref):
    n = pl.program_id(0)
    t = pl.program_id(1)
    x = x_ref[0]
    e = emb_ref[0]
    scores = lax.dot_general(
        x, e, (((1,), (1,)), ((), ())),
        preferred_element_type=jnp.float32)
    xn = jnp.sum(x * x, axis=1, keepdims=True)
    to_add = en_ref[0] + xn
    dist = to_add - 2.0 * scores
    mn = jnp.min(dist, axis=1, keepdims=True)
    iota_f = lax.broadcasted_iota(jnp.int32, (_TB, _M), 1).astype(jnp.float32)
    idxf = jnp.min(jnp.where(dist == mn, iota_f, float(_M)),
                   axis=1, keepdims=True)
    idx = idxf.astype(jnp.int32)
    idx_ref[...] = idx.reshape(_TB // _L, 1, _L, 1)
    fidx_ref[...] = (idx + n * _M).reshape(1, _TB // _CH, _CH)

    lsum = jnp.sum(mn)

    @pl.when((n == 0) & (t == 0))
    def _():
        loss_ref[...] = jnp.zeros((1, 1), jnp.float32)

    loss_ref[...] = loss_ref[...] + lsum

    @pl.when((n == _N - 1) & (t == _NTB - 1))
    def _():
        loss_ref[...] = loss_ref[...] * (_COMMIT / float(_NT * _D))


def _dist_argmin(x_flat, embedding, e_norm):
    out_shapes = [
        jax.ShapeDtypeStruct((_B, _N, _L, 1), jnp.int32),
        jax.ShapeDtypeStruct((_N * _NTB, _TB // _CH, _CH), jnp.int32),
        jax.ShapeDtypeStruct((1, 1), jnp.float32),
    ]
    return pl.pallas_call(
        _dist_argmin_body,
        grid=(_N, _NTB),
        in_specs=[
            pl.BlockSpec((1, _TB, _D), lambda n, t: (n, t, 0)),
            pl.BlockSpec((1, _M, _D), lambda n, t: (n, 0, 0)),
            pl.BlockSpec((1, 1, _M), lambda n, t: (n, 0, 0)),
        ],
        out_specs=[
            pl.BlockSpec((_TB // _L, 1, _L, 1), lambda n, t: (t, n, 0, 0)),
            pl.BlockSpec((1, _TB // _CH, _CH), lambda n, t: (n * _NTB + t, 0, 0)),
            pl.BlockSpec((1, 1), lambda n, t: (0, 0)),
        ],
        out_shape=out_shapes,
        compiler_params=pltpu.CompilerParams(
            dimension_semantics=("arbitrary", "arbitrary")),
    )(x_flat, embedding, e_norm)


def _sc_gather_scatter_body(idx_hbm, x_hbm, emb_hbm, zero_hbm, onesz_hbm,
                            q_out, dw_out, cnt_out,
                            idx_v, qrows, qrows2, xrows, xrows2, ones_v,
                            dwsh, csh, gsem, xsem):
    c = lax.axis_index("c")
    s = lax.axis_index("s")
    gbase = pl.multiple_of(c * (_NM // _NC) + s * (_NM // (_NC * _NS)), 8)
    pltpu.sync_copy(zero_hbm, dwsh.at[pl.ds(gbase, _NM // (_NC * _NS))])
    pltpu.sync_copy(onesz_hbm.at[pl.ds(_CH, _CH)], csh.at[pl.ds(gbase, _NM // (_NC * _NS))])
    pltpu.sync_copy(onesz_hbm.at[pl.ds(0, _CH)], ones_v)
    plsc.subcore_barrier()

    base = c * (_NT // _NC) + s * _ROWS_PER_TILE
    pltpu.sync_copy(idx_hbm.at[pl.ds(pl.multiple_of(base // _CH, 8), _NCHUNK)],
                    idx_v)
    qb = (qrows, qrows2)
    xb = (xrows, xrows2)
    hg = pltpu.async_copy(emb_hbm.at[idx_v.at[0]], qb[0], gsem)
    hx = pltpu.async_copy(x_hbm.at[pl.ds(pl.multiple_of(base, 8), _CH)],
                          xb[0], xsem)
    for j in range(_NCHUNK):
        cur = j % 2
        tok = pl.multiple_of(base + j * _CH, 8)
        if j + 1 < _NCHUNK:
            tok1 = pl.multiple_of(base + (j + 1) * _CH, 8)
            hg_n = pltpu.async_copy(emb_hbm.at[idx_v.at[j + 1]],
                                    qb[1 - cur], gsem)
            hx_n = pltpu.async_copy(x_hbm.at[pl.ds(tok1, _CH)],
                                    xb[1 - cur], xsem)
        hg.wait()
        pltpu.sync_copy(qb[cur], q_out.at[pl.ds(tok, _CH)])
        hx.wait()
        pltpu.sync_copy(xb[cur], dwsh.at[idx_v.at[j]], add=True)
        pltpu.sync_copy(ones_v, csh.at[idx_v.at[j]], add=True)
        if j + 1 < _NCHUNK:
            hg, hx = hg_n, hx_n
    plsc.subcore_barrier()
    pltpu.sync_copy(dwsh.at[pl.ds(gbase, _NM // (_NC * _NS))],
                    dw_out.at[pl.ds(gbase, _NM // (_NC * _NS))])
    pltpu.sync_copy(csh.at[pl.ds(gbase, _NM // (_NC * _NS))],
                    cnt_out.at[pl.ds(gbase, _NM // (_NC * _NS))])


_sc_gather_scatter = functools.partial(
    pl.kernel,
    _sc_gather_scatter_body,
    out_type=[
        jax.ShapeDtypeStruct((_NT, _D), jnp.float32),
        jax.ShapeDtypeStruct((_NM, _D), jnp.float32),
        jax.ShapeDtypeStruct((_NM, 16), jnp.float32),
    ],
    mesh=plsc.VectorSubcoreMesh(core_axis_name="c", subcore_axis_name="s"),
    compiler_params=pltpu.CompilerParams(use_tc_tiling_on_sc=False),
    scratch_types=[
        pltpu.VMEM((_NCHUNK, _CH), jnp.int32),
        pltpu.VMEM((_CH, _D), jnp.float32),
        pltpu.VMEM((_CH, _D), jnp.float32),
        pltpu.VMEM((_CH, _D), jnp.float32),
        pltpu.VMEM((_CH, _D), jnp.float32),
        pltpu.VMEM((_CH, 16), jnp.float32),
        pltpu.VMEM_SHARED((_NM, _D), jnp.float32),
        pltpu.VMEM_SHARED((_NM, 16), jnp.float32),
        pltpu.SemaphoreType.DMA,
        pltpu.SemaphoreType.DMA,
    ],
)()


def _sc_gather2_body(idx_hbm, emb_hbm, q_out, idx_v, qrows, qrows2, sem):
    c = lax.axis_index("c")
    s = lax.axis_index("s")
    base = c * (_NT // _NC) + s * _ROWS_PER_TILE
    pltpu.sync_copy(idx_hbm.at[pl.ds(pl.multiple_of(base // _CH, 8), _NCHUNK)],
                    idx_v)
    qb = (qrows, qrows2)
    hg = pltpu.async_copy(emb_hbm.at[idx_v.at[0]], qb[0], sem)
    for j in range(_NCHUNK):
        cur = j % 2
        tok = pl.multiple_of(base + j * _CH, 8)
        if j + 1 < _NCHUNK:
            hg_n = pltpu.async_copy(emb_hbm.at[idx_v.at[j + 1]],
                                    qb[1 - cur], sem)
        hg.wait()
        pltpu.sync_copy(qb[cur], q_out.at[pl.ds(tok, _CH)])
        if j + 1 < _NCHUNK:
            hg = hg_n


_sc_gather2 = functools.partial(
    pl.kernel,
    _sc_gather2_body,
    out_type=jax.ShapeDtypeStruct((_NT, _D), jnp.float32),
    mesh=plsc.VectorSubcoreMesh(core_axis_name="c", subcore_axis_name="s"),
    compiler_params=pltpu.CompilerParams(use_tc_tiling_on_sc=False),
    scratch_types=[
        pltpu.VMEM((_NCHUNK, _CH), jnp.int32),
        pltpu.VMEM((_CH, _D), jnp.float32),
        pltpu.VMEM((_CH, _D), jnp.float32),
        pltpu.SemaphoreType.DMA,
    ],
)()


def _ema_body(cnt_ref, ec_ref, w_ref, dw_ref, nemb_ref, perp_ref):
    cnt = jnp.sum(cnt_ref[...], axis=2) * (1.0 / 16.0)
    ec = ec_ref[...].astype(jnp.float32)
    dc = _DECAY * ec + (1.0 - _DECAY) * cnt
    nsum = jnp.sum(dc, axis=1, keepdims=True)
    nec = (dc + _EPS) / (nsum + _M * _EPS) * nsum
    new_w = _DECAY * w_ref[...] + (1.0 - _DECAY) * dw_ref[...]
    nemb_ref[...] = new_w / nec[:, :, None]
    p = cnt * (1.0 / _T)
    ent = -jnp.sum(p * jnp.log(p + 1e-10), axis=1, keepdims=True)
    perp_ref[...] = jnp.broadcast_to(jnp.sum(jnp.exp(ent)), (1, 1))


def _ema(counts3, ema_count, ema_weight, dw):
    return pl.pallas_call(
        _ema_body,
        out_shape=[
            jax.ShapeDtypeStruct((_N, _M, _D), jnp.float32),
            jax.ShapeDtypeStruct((1, 1), jnp.float32),
        ],
    )(counts3, ema_count, ema_weight, dw)


def kernel(x, embedding, ema_weight, ema_count):
    bs = x.shape[0]
    xr = x.reshape(bs, _N, _D, _L)
    x_flat = jnp.transpose(xr, (1, 0, 3, 2)).reshape(_N, bs * _L, _D)
    e_norm = jnp.sum(embedding ** 2, axis=2)[:, None, :]

    indices_out, fidx3, loss2 = _dist_argmin(x_flat, embedding, e_norm)
    idx2d = fidx3.reshape(_NT // _CH, _CH)

    emb_flat = embedding.reshape(_NM, _D)
    x_rows = x_flat.reshape(_NT, _D)
    zeros_tile = jnp.zeros((_NM // (_NC * _NS), _D), jnp.float32)
    onesz = jnp.concatenate([jnp.ones((_CH, 16), jnp.float32),
                             jnp.zeros((_CH, 16), jnp.float32)], axis=0)
    quant, dw, cnt16 = _sc_gather_scatter(idx2d, x_rows, emb_flat, zeros_tile,
                                          onesz)

    new_emb, perp2 = _ema(cnt16.reshape(_N, _M, 16), ema_count, ema_weight,
                          dw.reshape(_N, _M, _D))
    eq_rows = _sc_gather2(idx2d, new_emb.reshape(_NM, _D))

    loss = loss2.reshape(())
    perplexity = perp2.reshape(())
    z_q = jnp.transpose(quant.reshape(_N, bs, _L, _D), (1, 0, 3, 2)).reshape(bs, _N * _D * _L)
    encodings_q = jnp.transpose(eq_rows.reshape(_N, bs, _L, _D), (1, 0, 3, 2)).reshape(bs, _N * _D, _L, 1)
    return (z_q, loss, perplexity, indices_out, encodings_q)
